# baseline (speedup 1.0000x reference)
import functools

import jax
import jax.numpy as jnp
from jax import lax
from jax.experimental import pallas as pl
from jax.experimental.pallas import tpu as pltpu
from jax.experimental.pallas import tpu_sc as plsc

_B, _T, _D = 4, 4096, 1024
_NC, _NS, _L = 2, 16, 16
_NW = _NC * _NS
_ROWS_W = _T // _NW
_CT = 32
_NCH = _ROWS_W // _CT
_CHUNK = _CT * _D
_UNROLL = 8

_mesh = plsc.VectorSubcoreMesh(core_axis_name="c", subcore_axis_name="s")


@functools.partial(
    pl.kernel,
    out_type=jax.ShapeDtypeStruct((_B * _T * _D,), jnp.float32),
    mesh=_mesh,
    scratch_types=[
        pltpu.VMEM((_CHUNK,), jnp.float32),
        pltpu.VMEM((_CHUNK,), jnp.float32),
    ],
)
def _sc_add(x_hbm, pos_hbm, out_hbm, pbuf, wbuf):
    wid = lax.axis_index("s") * _NC + lax.axis_index("c")
    base = wid * _ROWS_W * _D
    for c in range(_NCH):
        pos_off = base + c * _CHUNK
        pltpu.sync_copy(pos_hbm.at[pl.ds(pos_off, _CHUNK)], pbuf)
        for b in range(_B):
            x_off = b * _T * _D + pos_off
            pltpu.sync_copy(x_hbm.at[pl.ds(x_off, _CHUNK)], wbuf)

            def _body(i, _):
                for u in range(_UNROLL):
                    s = pl.ds((i * _UNROLL + u) * _L, _L)
                    wbuf[s] = wbuf[s] + pbuf[s]
                return 0

            lax.fori_loop(0, _CHUNK // (_L * _UNROLL), _body, 0)
            pltpu.sync_copy(wbuf, out_hbm.at[pl.ds(x_off, _CHUNK)])


def kernel(x, pos_table):
    b, t, d = x.shape
    out = _sc_add(x.reshape(-1), pos_table[:t].reshape(-1))
    return out.reshape(b, t, d)


# speedup vs baseline: 1.1543x; 1.1543x over previous
import functools

import jax
import jax.numpy as jnp
from jax import lax
from jax.experimental import pallas as pl
from jax.experimental.pallas import tpu as pltpu
from jax.experimental.pallas import tpu_sc as plsc

_B, _T, _D = 4, 4096, 1024
_NC, _NS, _L = 2, 16, 16
_NW = _NC * _NS
_ROWS_W = _T // _NW
_CT = 16
_NCH = _ROWS_W // _CT
_CHUNK = _CT * _D
_NBUF = 3
_UNROLL = 8
_STEPS = [(c, b) for c in range(_NCH) for b in range(_B)]

_mesh = plsc.VectorSubcoreMesh(core_axis_name="c", subcore_axis_name="s")


@functools.partial(
    pl.kernel,
    out_type=jax.ShapeDtypeStruct((_B * _T * _D,), jnp.float32),
    mesh=_mesh,
    scratch_types=[
        [pltpu.VMEM((_CHUNK,), jnp.float32) for _ in range(_NBUF)],
        [pltpu.VMEM((_CHUNK,), jnp.float32) for _ in range(2)],
        [pltpu.SemaphoreType.DMA for _ in range(_NBUF)],
        [pltpu.SemaphoreType.DMA for _ in range(_NBUF)],
        [pltpu.SemaphoreType.DMA for _ in range(2)],
    ],
)
def _sc_add(x_hbm, pos_hbm, out_hbm, wb, pb, sin, sout, spos):
    wid = lax.axis_index("s") * _NC + lax.axis_index("c")
    base = wid * _ROWS_W * _D

    in_cp = [None] * len(_STEPS)
    out_cp = [None] * len(_STEPS)
    pos_cp = [None] * _NCH

    def issue_pos(c):
        pos_cp[c] = pltpu.async_copy(
            pos_hbm.at[pl.ds(base + c * _CHUNK, _CHUNK)], pb[c % 2], spos[c % 2]
        )

    def issue_in(s):
        c, b = _STEPS[s]
        if s >= _NBUF:
            out_cp[s - _NBUF].wait()
        off = b * _T * _D + base + c * _CHUNK
        in_cp[s] = pltpu.async_copy(
            x_hbm.at[pl.ds(off, _CHUNK)], wb[s % _NBUF], sin[s % _NBUF]
        )

    issue_pos(0)
    if _NCH > 1:
        issue_pos(1)
    for s in range(_NBUF):
        issue_in(s)

    for s, (c, b) in enumerate(_STEPS):
        k = s % _NBUF
        w, p = wb[k], pb[c % 2]
        in_cp[s].wait()
        if b == 0:
            pos_cp[c].wait()

        @plsc.parallel_loop(0, _CHUNK // (_L * _UNROLL))
        def _add(i):
            for u in range(_UNROLL):
                sl = pl.ds((i * _UNROLL + u) * _L, _L)
                w[sl] = w[sl] + p[sl]

        if b == _B - 1 and c + 2 < _NCH:
            issue_pos(c + 2)
        off = b * _T * _D + base + c * _CHUNK
        out_cp[s] = pltpu.async_copy(w, out_hbm.at[pl.ds(off, _CHUNK)], sout[k])
        if s + _NBUF < len(_STEPS):
            issue_in(s + _NBUF)

    for s in range(len(_STEPS) - _NBUF, len(_STEPS)):
        out_cp[s].wait()


def kernel(x, pos_table):
    b, t, d = x.shape
    out = _sc_add(x.reshape(-1), pos_table[:t].reshape(-1))
    return out.reshape(b, t, d)


# speedup vs baseline: 1.2262x; 1.0623x over previous
import functools

import jax
import jax.numpy as jnp
from jax import lax
from jax.experimental import pallas as pl
from jax.experimental.pallas import tpu as pltpu
from jax.experimental.pallas import tpu_sc as plsc

_B, _T, _D = 4, 4096, 1024
_NC, _NS, _L = 2, 16, 16
_NW = _NC * _NS
_ROWS_W = _T // _NW
_CT = 16
_NCH = _ROWS_W // _CT
_CHUNK = _CT * _D
_NBUF = 3
_UNROLL = 8
_STEPS = [(c, b) for c in range(_NCH) for b in range(_B)]

_mesh = plsc.VectorSubcoreMesh(core_axis_name="c", subcore_axis_name="s")


@functools.partial(
    pl.kernel,
    out_type=jax.ShapeDtypeStruct((_B * _T * _D,), jnp.float32),
    mesh=_mesh,
    scratch_types=[
        [pltpu.VMEM((_CHUNK,), jnp.float32) for _ in range(_NBUF)],
        [pltpu.VMEM((_CHUNK,), jnp.float32) for _ in range(2)],
        [pltpu.SemaphoreType.DMA for _ in range(_NBUF)],
        [pltpu.SemaphoreType.DMA for _ in range(_NBUF)],
        [pltpu.SemaphoreType.DMA for _ in range(2)],
    ],
)
def _sc_add(x_hbm, pos_hbm, out_hbm, wb, pb, sin, sout, spos):
    wid = lax.axis_index("s") * _NC + lax.axis_index("c")
    base = wid * _ROWS_W * _D

    in_cp = [None] * len(_STEPS)
    out_cp = [None] * len(_STEPS)
    pos_cp = [None] * _NCH

    def issue_pos(c):
        pos_cp[c] = pltpu.async_copy(
            pos_hbm.at[pl.ds(base + c * _CHUNK, _CHUNK)], pb[c % 2], spos[c % 2]
        )

    def issue_in(s):
        c, b = _STEPS[s]
        if s >= _NBUF:
            out_cp[s - _NBUF].wait()
        off = b * _T * _D + base + c * _CHUNK
        in_cp[s] = pltpu.async_copy(
            x_hbm.at[pl.ds(off, _CHUNK)], wb[s % _NBUF], sin[s % _NBUF]
        )

    issue_pos(0)
    if _NCH > 1:
        issue_pos(1)
    for s in range(_NBUF):
        issue_in(s)

    for s, (c, b) in enumerate(_STEPS):
        k = s % _NBUF
        w, p = wb[k], pb[c % 2]
        in_cp[s].wait()
        if b == 0:
            pos_cp[c].wait()

        if False:
            @plsc.parallel_loop(0, _CHUNK // (_L * _UNROLL))
            def _add(i):
                for u in range(_UNROLL):
                    sl = pl.ds((i * _UNROLL + u) * _L, _L)
                    w[sl] = w[sl] + p[sl]

        if b == _B - 1 and c + 2 < _NCH:
            issue_pos(c + 2)
        off = b * _T * _D + base + c * _CHUNK
        out_cp[s] = pltpu.async_copy(w, out_hbm.at[pl.ds(off, _CHUNK)], sout[k])
        if s + _NBUF < len(_STEPS):
            issue_in(s + _NBUF)

    for s in range(len(_STEPS) - _NBUF, len(_STEPS)):
        out_cp[s].wait()


def kernel(x, pos_table):
    b, t, d = x.shape
    out = _sc_add(x.reshape(-1), pos_table[:t].reshape(-1))
    return out.reshape(b, t, d)


# speedup vs baseline: 3.1288x; 2.5515x over previous
import functools

import jax
import jax.numpy as jnp
from jax import lax
from jax.experimental import pallas as pl
from jax.experimental.pallas import tpu as pltpu
from jax.experimental.pallas import tpu_sc as plsc

_B, _T, _D = 4, 4096, 1024
_NC, _NS, _L = 2, 16, 16
_NW = _NC * _NS
_ROWS_W = _T // _NW
_CT = 16
_NCH = _ROWS_W // _CT
_NBUF = 3
_UNROLL = 8
_BLKS_ROW = _D // (_L * _UNROLL)
_STEPS = [(c, b) for c in range(_NCH) for b in range(_B)]

_mesh = plsc.VectorSubcoreMesh(core_axis_name="c", subcore_axis_name="s")


@functools.partial(
    pl.kernel,
    out_type=jax.ShapeDtypeStruct((_B, _T, _D), jnp.float32),
    mesh=_mesh,
    scratch_types=[
        [pltpu.VMEM((_CT, _D), jnp.float32) for _ in range(_NBUF)],
        [pltpu.VMEM((_CT, _D), jnp.float32) for _ in range(2)],
        [pltpu.SemaphoreType.DMA for _ in range(_NBUF)],
        [pltpu.SemaphoreType.DMA for _ in range(_NBUF)],
        [pltpu.SemaphoreType.DMA for _ in range(2)],
    ],
)
def _sc_add(x_hbm, pos_hbm, out_hbm, wb, pb, sin, sout, spos):
    wid = lax.axis_index("s") * _NC + lax.axis_index("c")
    t_base = wid * _ROWS_W

    in_cp = [None] * len(_STEPS)
    out_cp = [None] * len(_STEPS)
    pos_cp = [None] * _NCH

    def issue_pos(c):
        pos_cp[c] = pltpu.async_copy(
            pos_hbm.at[pl.ds(t_base + c * _CT, _CT)], pb[c % 2], spos[c % 2]
        )

    def issue_in(s):
        c, b = _STEPS[s]
        if s >= _NBUF:
            out_cp[s - _NBUF].wait()
        in_cp[s] = pltpu.async_copy(
            x_hbm.at[b, pl.ds(t_base + c * _CT, _CT)], wb[s % _NBUF], sin[s % _NBUF]
        )

    issue_pos(0)
    if _NCH > 1:
        issue_pos(1)
    for s in range(_NBUF):
        issue_in(s)

    for s, (c, b) in enumerate(_STEPS):
        k = s % _NBUF
        w, p = wb[k], pb[c % 2]
        in_cp[s].wait()
        if b == 0:
            pos_cp[c].wait()

        @plsc.parallel_loop(0, _CT * _BLKS_ROW)
        def _add(i):
            r = i // _BLKS_ROW
            j = i % _BLKS_ROW
            for u in range(_UNROLL):
                sl = pl.ds((j * _UNROLL + u) * _L, _L)
                w[r, sl] = w[r, sl] + p[r, sl]

        if b == _B - 1 and c + 2 < _NCH:
            issue_pos(c + 2)
        out_cp[s] = pltpu.async_copy(
            w, out_hbm.at[b, pl.ds(t_base + c * _CT, _CT)], sout[k]
        )
        if s + _NBUF < len(_STEPS):
            issue_in(s + _NBUF)

    for s in range(len(_STEPS) - _NBUF, len(_STEPS)):
        out_cp[s].wait()


def kernel(x, pos_table):
    return _sc_add(x, pos_table)


# speedup vs baseline: 3.1841x; 1.0177x over previous
import functools

import jax
import jax.numpy as jnp
from jax import lax
from jax.experimental import pallas as pl
from jax.experimental.pallas import tpu as pltpu
from jax.experimental.pallas import tpu_sc as plsc

_B, _T, _D = 4, 4096, 1024
_NC, _NS, _L = 2, 16, 16
_NW = _NC * _NS
_ROWS_W = _T // _NW
_CT = 16
_NCH = _ROWS_W // _CT
_NBUF = 5
_UNROLL = 8
_BLKS_ROW = _D // (_L * _UNROLL)
_STEPS = [(c, b) for c in range(_NCH) for b in range(_B)]

_mesh = plsc.VectorSubcoreMesh(core_axis_name="c", subcore_axis_name="s")


@functools.partial(
    pl.kernel,
    out_type=jax.ShapeDtypeStruct((_B, _T, _D), jnp.float32),
    mesh=_mesh,
    scratch_types=[
        [pltpu.VMEM((_CT, _D), jnp.float32) for _ in range(_NBUF)],
        [pltpu.VMEM((_CT, _D), jnp.float32) for _ in range(2)],
        [pltpu.SemaphoreType.DMA for _ in range(_NBUF)],
        [pltpu.SemaphoreType.DMA for _ in range(_NBUF)],
        [pltpu.SemaphoreType.DMA for _ in range(2)],
    ],
)
def _sc_add(x_hbm, pos_hbm, out_hbm, wb, pb, sin, sout, spos):
    wid = lax.axis_index("s") * _NC + lax.axis_index("c")
    t_base = wid * _ROWS_W

    in_cp = [None] * len(_STEPS)
    out_cp = [None] * len(_STEPS)
    pos_cp = [None] * _NCH

    def issue_pos(c):
        pos_cp[c] = pltpu.async_copy(
            pos_hbm.at[pl.ds(t_base + c * _CT, _CT)], pb[c % 2], spos[c % 2]
        )

    def issue_in(s):
        c, b = _STEPS[s]
        if s >= _NBUF:
            out_cp[s - _NBUF].wait()
        in_cp[s] = pltpu.async_copy(
            x_hbm.at[b, pl.ds(t_base + c * _CT, _CT)], wb[s % _NBUF], sin[s % _NBUF]
        )

    issue_pos(0)
    if _NCH > 1:
        issue_pos(1)
    for s in range(_NBUF):
        issue_in(s)

    for s, (c, b) in enumerate(_STEPS):
        k = s % _NBUF
        w, p = wb[k], pb[c % 2]
        in_cp[s].wait()
        if b == 0:
            pos_cp[c].wait()

        @plsc.parallel_loop(0, _CT * _BLKS_ROW)
        def _add(i):
            r = i // _BLKS_ROW
            j = i % _BLKS_ROW
            for u in range(_UNROLL):
                sl = pl.ds((j * _UNROLL + u) * _L, _L)
                w[r, sl] = w[r, sl] + p[r, sl]

        if b == _B - 1 and c + 2 < _NCH:
            issue_pos(c + 2)
        out_cp[s] = pltpu.async_copy(
            w, out_hbm.at[b, pl.ds(t_base + c * _CT, _CT)], sout[k]
        )
        if s + _NBUF < len(_STEPS):
            issue_in(s + _NBUF)

    for s in range(len(_STEPS) - _NBUF, len(_STEPS)):
        out_cp[s].wait()


def kernel(x, pos_table):
    return _sc_add(x, pos_table)
